# baseline (speedup 1.0000x reference)
import functools

import jax
import jax.numpy as jnp
from jax import lax
from jax.experimental import pallas as pl
from jax.experimental.pallas import tpu as pltpu
from jax.experimental.pallas import tpu_sc as plsc

_BETA = 0.9
_K1 = 10.0
_SUPPRESSION_EPS = 10.0

_B = 16384
_C = 1000
_N = 1_000_000

_BB = 2048
_NW = 32
_SPT = _B // _NW
_VSTEP = 16


def _loss_body(l_ref, tgt_ref, loss_ref):
    i = pl.program_id(0)
    x = l_ref[0].astype(jnp.float32)
    t = tgt_ref[0, pl.ds(i * _BB, _BB)]
    m = jnp.max(x, axis=1)
    e = jnp.exp(x - m[:, None])
    s = jnp.sum(e, axis=1)
    col = lax.broadcasted_iota(jnp.int32, (1, _C), 1)
    tl = jnp.sum(jnp.where(col == t[:, None], x, 0.0), axis=1)
    loss_ref[0, pl.ds(i * _BB, _BB)] = jnp.log(s) + m - tl


def _compute_loss(logits, targets):
    l3 = logits.astype(jnp.bfloat16).reshape(_B // _BB, _BB, _C)
    tgt2 = targets.reshape(1, _B)
    loss2 = pl.pallas_call(
        _loss_body,
        grid=(_B // _BB,),
        in_specs=[
            pl.BlockSpec((1, _BB, _C), lambda i: (i, 0, 0)),
            pl.BlockSpec((1, _B), lambda i: (0, 0)),
        ],
        out_specs=pl.BlockSpec((1, _B), lambda i: (0, 0)),
        out_shape=jax.ShapeDtypeStruct((1, _B), jnp.float32),
    )(l3, tgt2)
    return loss2.reshape(_B)


@functools.partial(
    pl.kernel,
    out_type=(),
    mesh=plsc.VectorSubcoreMesh(core_axis_name="c", subcore_axis_name="s"),
    scratch_types=[
        pltpu.VMEM((_SPT,), jnp.int32),
        pltpu.VMEM((_SPT,), jnp.float32),
        pltpu.VMEM((_SPT,), jnp.float32),
        pltpu.VMEM((_SPT,), jnp.float32),
        pltpu.VMEM((_SPT,), jnp.float32),
        pltpu.VMEM((_SPT,), jnp.float32),
        pltpu.VMEM((2 * _VSTEP,), jnp.float32),
        pltpu.SemaphoreType.DMA,
        pltpu.SemaphoreType.DMA,
        pltpu.SemaphoreType.DMA,
        pltpu.SemaphoreType.DMA,
        pltpu.SemaphoreType.DMA,
        pltpu.SemaphoreType.DMA,
    ],
)
def _sc_kernel(exp_hbm, idx_hbm, loss_hbm, dpm_hbm, s_hbm,
               buf_hbm, out1_hbm,
               idx_v, g_v, new_v, loss_v, dpm_v, out1_v, s_v,
               sem_i, sem_l, sem_d, sem_s, sem_g, sem_o):
    core = lax.axis_index("c")
    tid = lax.axis_index("s")
    wid = tid * 2 + core
    base = wid * _SPT

    pltpu.async_copy(idx_hbm.at[pl.ds(base, _SPT)], idx_v, sem_i)
    pltpu.async_copy(loss_hbm.at[pl.ds(base, _SPT)], loss_v, sem_l)
    pltpu.async_copy(dpm_hbm.at[pl.ds(base, _SPT)], dpm_v, sem_d)
    pltpu.async_copy(s_hbm, s_v, sem_s)

    pltpu.make_async_copy(idx_hbm.at[pl.ds(base, _SPT)], idx_v, sem_i).wait()
    pltpu.async_copy(exp_hbm.at[idx_v], g_v, sem_g)

    pltpu.make_async_copy(loss_hbm.at[pl.ds(base, _SPT)], loss_v,
                          sem_l).wait()
    pltpu.make_async_copy(dpm_hbm.at[pl.ds(base, _SPT)], dpm_v, sem_d).wait()
    pltpu.make_async_copy(s_hbm, s_v, sem_s).wait()
    pltpu.make_async_copy(exp_hbm.at[idx_v], g_v, sem_g).wait()

    s1 = s_v[pl.ds(0, _VSTEP)]
    s2 = s_v[pl.ds(_VSTEP, _VSTEP)]
    for k in range(_SPT // _VSTEP):
        sl = pl.ds(k * _VSTEP, _VSTEP)
        nv = g_v[sl] * _BETA + loss_v[sl] * (1.0 - _BETA)
        new_v[sl] = nv
        out1_v[sl] = (nv * s1 - s2) / dpm_v[sl]

    pltpu.async_copy(out1_v, out1_hbm.at[pl.ds(base, _SPT)], sem_o)
    pltpu.async_copy(new_v, buf_hbm.at[idx_v], sem_g)

    pltpu.make_async_copy(out1_v, out1_hbm.at[pl.ds(base, _SPT)],
                          sem_o).wait()
    pltpu.make_async_copy(new_v, buf_hbm.at[idx_v], sem_g).wait()


def kernel(logits, targets, data_parameter_minibatch, exp_avg, index_dataset, epoch):
    loss = _compute_loss(logits, targets.astype(jnp.int32))

    ep = jnp.asarray(epoch, jnp.float32)
    es = jnp.where(ep < _SUPPRESSION_EPS, (ep + 1.0) / 10.0, 1.0)
    bias_cor = 1.0 - jnp.power(_BETA, ep + 1.0)
    s1 = es / bias_cor
    s2 = _K1 * es
    s_arr = jnp.concatenate([
        jnp.full((_VSTEP,), s1, jnp.float32),
        jnp.full((_VSTEP,), s2, jnp.float32),
    ])

    buf = jax.new_ref(exp_avg)
    out1 = jax.new_ref(jnp.zeros((_B,), jnp.float32))
    _sc_kernel(exp_avg, index_dataset.astype(jnp.int32), loss,
               data_parameter_minibatch, s_arr, buf, out1)
    return out1[...], buf[...]


# speedup vs baseline: 1.0596x; 1.0596x over previous
import functools

import jax
import jax.numpy as jnp
from jax import lax
from jax.experimental import pallas as pl
from jax.experimental.pallas import tpu as pltpu
from jax.experimental.pallas import tpu_sc as plsc

_BETA = 0.9
_K1 = 10.0
_SUPPRESSION_EPS = 10.0

_B = 16384
_C = 1000
_N = 1_000_000

_BB = 2048
_NW = 32
_SPT = _B // _NW
_VSTEP = 16


def _loss_body(l_ref, tgt_ref, loss_ref):
    i = pl.program_id(0)
    x = l_ref[0]
    t = tgt_ref[0, pl.ds(i * _BB, _BB)]
    m = jnp.max(x, axis=1)
    e = jnp.exp(x - m[:, None])
    s = jnp.sum(e, axis=1)
    col = lax.broadcasted_iota(jnp.int32, (1, _C), 1)
    tl = jnp.sum(jnp.where(col == t[:, None], x, 0.0), axis=1)
    loss_ref[0, pl.ds(i * _BB, _BB)] = jnp.log(s) + m - tl


def _compute_loss(logits, targets):
    l3 = logits.reshape(_B // _BB, _BB, _C)
    tgt2 = targets.reshape(1, _B)
    loss2 = pl.pallas_call(
        _loss_body,
        grid=(_B // _BB,),
        in_specs=[
            pl.BlockSpec((1, _BB, _C), lambda i: (i, 0, 0)),
            pl.BlockSpec((1, _B), lambda i: (0, 0)),
        ],
        out_specs=pl.BlockSpec((1, _B), lambda i: (0, 0)),
        out_shape=jax.ShapeDtypeStruct((1, _B), jnp.float32),
    )(l3, tgt2)
    return loss2.reshape(_B)


@functools.partial(
    pl.kernel,
    out_type=(),
    mesh=plsc.VectorSubcoreMesh(core_axis_name="c", subcore_axis_name="s"),
    scratch_types=[
        pltpu.VMEM((_SPT,), jnp.int32),
        pltpu.VMEM((_SPT,), jnp.float32),
        pltpu.VMEM((_SPT,), jnp.float32),
        pltpu.VMEM((_SPT,), jnp.float32),
        pltpu.VMEM((_SPT,), jnp.float32),
        pltpu.VMEM((_SPT,), jnp.float32),
        pltpu.VMEM((2 * _VSTEP,), jnp.float32),
        pltpu.SemaphoreType.DMA,
        pltpu.SemaphoreType.DMA,
        pltpu.SemaphoreType.DMA,
        pltpu.SemaphoreType.DMA,
        pltpu.SemaphoreType.DMA,
        pltpu.SemaphoreType.DMA,
    ],
)
def _sc_kernel(exp_hbm, idx_hbm, loss_hbm, dpm_hbm, s_hbm,
               buf_hbm, out1_hbm,
               idx_v, g_v, new_v, loss_v, dpm_v, out1_v, s_v,
               sem_i, sem_l, sem_d, sem_s, sem_g, sem_o):
    core = lax.axis_index("c")
    tid = lax.axis_index("s")
    wid = tid * 2 + core
    base = wid * _SPT

    pltpu.async_copy(idx_hbm.at[pl.ds(base, _SPT)], idx_v, sem_i)
    pltpu.async_copy(loss_hbm.at[pl.ds(base, _SPT)], loss_v, sem_l)
    pltpu.async_copy(dpm_hbm.at[pl.ds(base, _SPT)], dpm_v, sem_d)
    pltpu.async_copy(s_hbm, s_v, sem_s)

    pltpu.make_async_copy(idx_hbm.at[pl.ds(base, _SPT)], idx_v, sem_i).wait()
    pltpu.async_copy(exp_hbm.at[idx_v], g_v, sem_g)

    pltpu.make_async_copy(loss_hbm.at[pl.ds(base, _SPT)], loss_v,
                          sem_l).wait()
    pltpu.make_async_copy(dpm_hbm.at[pl.ds(base, _SPT)], dpm_v, sem_d).wait()
    pltpu.make_async_copy(s_hbm, s_v, sem_s).wait()
    pltpu.make_async_copy(exp_hbm.at[idx_v], g_v, sem_g).wait()

    s1 = s_v[pl.ds(0, _VSTEP)]
    s2 = s_v[pl.ds(_VSTEP, _VSTEP)]
    for k in range(_SPT // _VSTEP):
        sl = pl.ds(k * _VSTEP, _VSTEP)
        nv = g_v[sl] * _BETA + loss_v[sl] * (1.0 - _BETA)
        new_v[sl] = nv
        out1_v[sl] = (nv * s1 - s2) / dpm_v[sl]

    pltpu.async_copy(out1_v, out1_hbm.at[pl.ds(base, _SPT)], sem_o)
    pltpu.async_copy(new_v, buf_hbm.at[idx_v], sem_g)

    pltpu.make_async_copy(out1_v, out1_hbm.at[pl.ds(base, _SPT)],
                          sem_o).wait()
    pltpu.make_async_copy(new_v, buf_hbm.at[idx_v], sem_g).wait()


def kernel(logits, targets, data_parameter_minibatch, exp_avg, index_dataset, epoch):
    loss = _compute_loss(logits, targets.astype(jnp.int32))

    ep = jnp.asarray(epoch, jnp.float32)
    es = jnp.where(ep < _SUPPRESSION_EPS, (ep + 1.0) / 10.0, 1.0)
    bias_cor = 1.0 - jnp.power(_BETA, ep + 1.0)
    s1 = es / bias_cor
    s2 = _K1 * es
    s_arr = jnp.concatenate([
        jnp.full((_VSTEP,), s1, jnp.float32),
        jnp.full((_VSTEP,), s2, jnp.float32),
    ])

    buf = jax.new_ref(exp_avg)
    out1 = jax.new_ref(jnp.zeros((_B,), jnp.float32))
    _sc_kernel(exp_avg, index_dataset.astype(jnp.int32), loss,
               data_parameter_minibatch, s_arr, buf, out1)
    return out1[...], buf[...]
